# Initial kernel scaffold; baseline (speedup 1.0000x reference)
import jax
import jax.numpy as jnp
from jax.experimental import pallas as pl


def kernel(x, edge_index, node_i, node_j, W_in, b_in, distmult, W1, b1, W2, b2):
    raise NotImplementedError("write your pallas kernel here")


# speedup vs baseline: 12.6245x; 12.6245x over previous
import functools

import jax
import jax.numpy as jnp
from jax import lax
from jax.experimental import pallas as pl
from jax.experimental.pallas import tpu as pltpu
from jax.experimental.pallas import tpu_sc as plsc

NC = 2
NS = 16
NW = NC * NS
EB = 128


def _mesh():
    return plsc.VectorSubcoreMesh(
        core_axis_name="c", subcore_axis_name="s", num_cores=NC, num_subcores=NS
    )


def _wid():
    return lax.axis_index("s") * NC + lax.axis_index("c")


def _degree_sc(dstb, nacc, rows_per_tile):
    nblk = dstb.shape[0]
    iters = (nblk + NW - 1) // NW
    zeros = jnp.zeros((nacc,), jnp.float32)

    @functools.partial(
        pl.kernel,
        out_type=jax.ShapeDtypeStruct((NC, nacc), jnp.float32),
        mesh=_mesh(),
        scratch_types=[
            pltpu.VMEM((1, EB), jnp.int32),
            pltpu.VMEM((EB,), jnp.float32),
            pltpu.VMEM_SHARED((nacc,), jnp.float32),
            pltpu.SemaphoreType.DMA,
        ],
    )
    def deg_kernel(dstb_hbm, zero_hbm, out_hbm, idx_v, ones_v, acc_sh, sem):
        c = lax.axis_index("c")
        sid = lax.axis_index("s")
        wid = _wid()
        for g in range(EB // 16):
            ones_v[pl.ds(g * 16, 16)] = jnp.ones((16,), jnp.float32)

        @pl.when(sid == 0)
        def _():
            pltpu.sync_copy(zero_hbm, acc_sh)

        plsc.subcore_barrier()

        def body(i, _):
            blk = i * NW + wid

            @pl.when(blk < nblk)
            def _():
                pltpu.sync_copy(dstb_hbm.at[blk], idx_v.at[0])
                pltpu.sync_copy(ones_v, acc_sh.at[idx_v.at[0]], add=True)

            return ()

        lax.fori_loop(0, iters, body, ())
        plsc.subcore_barrier()

        @pl.when(sid == 0)
        def _():
            pltpu.sync_copy(acc_sh, out_hbm.at[c])

    return deg_kernel(dstb, zeros)


def _segsum_sc(u, srcb, dstb, nacc, rows_per_tile):
    nblk = srcb.shape[0]
    H = u.shape[1]
    iters = (nblk + NW - 1) // NW
    zeros = jnp.zeros((rows_per_tile, H), jnp.float32)

    @functools.partial(
        pl.kernel,
        out_type=jax.ShapeDtypeStruct((NC, nacc, H), jnp.float32),
        mesh=_mesh(),
        scratch_types=[
            pltpu.VMEM((EB,), jnp.int32),
            pltpu.VMEM((1, EB), jnp.int32),
            pltpu.VMEM((EB, H), jnp.float32),
            pltpu.VMEM_SHARED((nacc, H), jnp.float32),
            pltpu.SemaphoreType.DMA,
        ],
    )
    def seg_kernel(u_hbm, srcb_hbm, dstb_hbm, zero_hbm, out_hbm,
                   sidx_v, didx_v, rows_v, acc_sh, sem):
        c = lax.axis_index("c")
        sid = lax.axis_index("s")
        wid = _wid()
        base = sid * rows_per_tile
        pltpu.sync_copy(zero_hbm, acc_sh.at[pl.ds(base, rows_per_tile)])
        plsc.subcore_barrier()

        def body(i, _):
            blk = i * NW + wid

            @pl.when(blk < nblk)
            def _():
                pltpu.sync_copy(srcb_hbm.at[blk], sidx_v)
                pltpu.sync_copy(dstb_hbm.at[blk], didx_v.at[0])
                pltpu.async_copy(u_hbm.at[sidx_v], rows_v, sem).wait()
                pltpu.sync_copy(rows_v, acc_sh.at[didx_v.at[0]], add=True)

            return ()

        lax.fori_loop(0, iters, body, ())
        plsc.subcore_barrier()
        pltpu.sync_copy(acc_sh.at[pl.ds(base, rows_per_tile)],
                        out_hbm.at[c, pl.ds(base, rows_per_tile)])

    return seg_kernel(u, srcb, dstb, zeros)


def _pairs_sc(a, h2, ib, jb):
    nblk = ib.shape[0]
    H = a.shape[1]
    G = H // 16
    iters = (nblk + NW - 1) // NW

    @functools.partial(
        pl.kernel,
        out_type=jax.ShapeDtypeStruct((nblk, EB, 16), jnp.float32),
        mesh=_mesh(),
        scratch_types=[
            pltpu.VMEM((EB,), jnp.int32),
            pltpu.VMEM((EB,), jnp.int32),
            pltpu.VMEM((EB, H), jnp.float32),
            pltpu.VMEM((EB, H), jnp.float32),
            pltpu.VMEM((EB, 16), jnp.float32),
            pltpu.SemaphoreType.DMA,
            pltpu.SemaphoreType.DMA,
        ],
    )
    def pair_kernel(a_hbm, h2_hbm, ib_hbm, jb_hbm, out_hbm,
                    iidx_v, jidx_v, va, vb, part_v, sema, semb):
        wid = _wid()

        def body(i, _):
            blk = i * NW + wid

            @pl.when(blk < nblk)
            def _():
                pltpu.sync_copy(ib_hbm.at[blk], iidx_v)
                pltpu.sync_copy(jb_hbm.at[blk], jidx_v)
                ca = pltpu.async_copy(a_hbm.at[iidx_v], va, sema)
                cb = pltpu.async_copy(h2_hbm.at[jidx_v], vb, semb)
                ca.wait()
                cb.wait()

                def pbody(p, _):
                    acc = va[p, pl.ds(0, 16)] * vb[p, pl.ds(0, 16)]
                    for g in range(1, G):
                        acc += va[p, pl.ds(g * 16, 16)] * vb[p, pl.ds(g * 16, 16)]
                    part_v[p, :] = acc
                    return ()

                lax.fori_loop(0, EB, pbody, ())
                pltpu.sync_copy(part_v, out_hbm.at[blk])

            return ()

        lax.fori_loop(0, iters, body, ())

    return pair_kernel(a, h2, ib, jb)


def _dense1_tc(x, deg2, WinT, b_in2, W1T):
    N, D = x.shape
    H = WinT.shape[1]
    BR = 1000
    grid = (N // BR,)

    def body(x_ref, deg_ref, winT_ref, b_ref, w1T_ref, out_ref):
        dinv = lax.rsqrt(deg_ref[:, 0] + deg_ref[:, 1] + 1.0)
        h0 = jnp.dot(x_ref[...], winT_ref[...], preferred_element_type=jnp.float32)
        h0 = jnp.maximum(h0 + b_ref[...], 0.0)
        u1 = jnp.dot(h0, w1T_ref[...], preferred_element_type=jnp.float32)
        out_ref[...] = u1 * dinv[:, None]

    return pl.pallas_call(
        body,
        grid=grid,
        in_specs=[
            pl.BlockSpec((BR, D), lambda i: (i, 0)),
            pl.BlockSpec((BR, 2), lambda i: (i, 0)),
            pl.BlockSpec((D, H), lambda i: (0, 0)),
            pl.BlockSpec((1, H), lambda i: (0, 0)),
            pl.BlockSpec((H, H), lambda i: (0, 0)),
        ],
        out_specs=pl.BlockSpec((BR, H), lambda i: (i, 0)),
        out_shape=jax.ShapeDtypeStruct((N, H), jnp.float32),
    )(x, deg2, WinT, b_in2, W1T)


def _dense2_tc(acc, u1, deg2, b12, W2T):
    N, H = u1.shape
    BR = 1000
    grid = (N // BR,)

    def body(acc_ref, u_ref, deg_ref, b_ref, w2T_ref, out_ref):
        dinv = lax.rsqrt(deg_ref[:, 0] + deg_ref[:, 1] + 1.0)
        s = acc_ref[0] + acc_ref[1] + u_ref[...]
        h1 = jnp.maximum(s * dinv[:, None] + b_ref[...], 0.0)
        u2 = jnp.dot(h1, w2T_ref[...], preferred_element_type=jnp.float32)
        out_ref[...] = u2 * dinv[:, None]

    return pl.pallas_call(
        body,
        grid=grid,
        in_specs=[
            pl.BlockSpec((2, BR, H), lambda i: (0, i, 0)),
            pl.BlockSpec((BR, H), lambda i: (i, 0)),
            pl.BlockSpec((BR, 2), lambda i: (i, 0)),
            pl.BlockSpec((1, H), lambda i: (0, 0)),
            pl.BlockSpec((H, H), lambda i: (0, 0)),
        ],
        out_specs=pl.BlockSpec((BR, H), lambda i: (i, 0)),
        out_shape=jax.ShapeDtypeStruct((N, H), jnp.float32),
    )(acc, u1, deg2, b12, W2T)


def _dense3_tc(acc, u2, deg2, b22, dm):
    N, H = u2.shape
    BR = 1000
    grid = (N // BR,)

    def body(acc_ref, u_ref, deg_ref, b_ref, dm_ref, h2_ref, a_ref):
        dinv = lax.rsqrt(deg_ref[:, 0] + deg_ref[:, 1] + 1.0)
        s = acc_ref[0] + acc_ref[1] + u_ref[...]
        h2 = s * dinv[:, None] + b_ref[...]
        h2_ref[...] = h2
        a_ref[...] = h2 * dm_ref[...]

    return pl.pallas_call(
        body,
        grid=grid,
        in_specs=[
            pl.BlockSpec((2, BR, H), lambda i: (0, i, 0)),
            pl.BlockSpec((BR, H), lambda i: (i, 0)),
            pl.BlockSpec((BR, 2), lambda i: (i, 0)),
            pl.BlockSpec((1, H), lambda i: (0, 0)),
            pl.BlockSpec((1, H), lambda i: (0, 0)),
        ],
        out_specs=[
            pl.BlockSpec((BR, H), lambda i: (i, 0)),
            pl.BlockSpec((BR, H), lambda i: (i, 0)),
        ],
        out_shape=[
            jax.ShapeDtypeStruct((N, H), jnp.float32),
            jax.ShapeDtypeStruct((N, H), jnp.float32),
        ],
    )(acc, u2, deg2, b22, dm)


def _reduce_tc(part):
    nblk, eb, L = part.shape
    BR = 16
    grid = (nblk // BR,)

    def body(p_ref, out_ref):
        out_ref[...] = jnp.sum(p_ref[...], axis=-1)

    return pl.pallas_call(
        body,
        grid=grid,
        in_specs=[pl.BlockSpec((BR, eb, L), lambda i: (i, 0, 0))],
        out_specs=pl.BlockSpec((BR, eb), lambda i: (i, 0)),
        out_shape=jax.ShapeDtypeStruct((nblk, eb), jnp.float32),
    )(part)


def kernel(x, edge_index, node_i, node_j, W_in, b_in, distmult, W1, b1, W2, b2):
    N, D = x.shape
    H = W_in.shape[0]
    E = edge_index.shape[1]
    P = node_i.shape[0]

    rows_per_tile = (-(-N // NS) + 7) // 8 * 8
    nacc = rows_per_tile * NS

    epad = (-E) % EB
    src = edge_index[0]
    dst = edge_index[1]
    if epad:
        src = jnp.concatenate([src, jnp.zeros((epad,), jnp.int32)])
        dst = jnp.concatenate([dst, jnp.full((epad,), nacc - 1, jnp.int32)])
    srcb = src.reshape(-1, EB)
    dstb = dst.reshape(-1, EB)

    deg2 = _degree_sc(dstb, nacc, rows_per_tile)[:, :N].T

    WinT = W_in.T
    W1T = W1.T
    W2T = W2.T
    b_in2 = b_in.reshape(1, H)
    b12 = b1.reshape(1, H)
    b22 = b2.reshape(1, H)

    u1 = _dense1_tc(x, deg2, WinT, b_in2, W1T)
    u1p = jnp.concatenate(
        [u1, jnp.zeros((nacc - N, H), jnp.float32)]) if nacc > N else u1
    acc1 = _segsum_sc(u1p, srcb, dstb, nacc, rows_per_tile)[:, :N, :]

    u2 = _dense2_tc(acc1, u1, deg2, b12, W2T)
    u2p = jnp.concatenate(
        [u2, jnp.zeros((nacc - N, H), jnp.float32)]) if nacc > N else u2
    acc2 = _segsum_sc(u2p, srcb, dstb, nacc, rows_per_tile)[:, :N, :]

    h2, a = _dense3_tc(acc2, u2, deg2, b22, distmult)

    ppad = (-P) % (EB * 16)
    ib = jnp.concatenate([node_i, jnp.zeros((ppad,), jnp.int32)]).reshape(-1, EB)
    jb = jnp.concatenate([node_j, jnp.zeros((ppad,), jnp.int32)]).reshape(-1, EB)

    part = _pairs_sc(a, h2, ib, jb)
    scores = _reduce_tc(part).reshape(-1)[:P]
    return scores
